# baseline (speedup 1.0000x reference)
import jax
import jax.numpy as jnp
from jax import lax
from jax.experimental import pallas as pl
from jax.experimental.pallas import tpu as pltpu
from jax.experimental.pallas import tpu_sc as plsc

VOCAB = 1000000
MAX_LEN = 200
D = 64
B = 1024
T = 200

NC = 2
NS = 16
NW = NC * NS
CH = 40
CPW = (B * T) // (NW * CH)
NBUF = 4
LANES = 16
VPR = D // LANES

VBLK = 4096
TBLK = 8


def _pack_body(tt_ref, out_ref):
    out_ref[:, pl.ds(0, D)] = tt_ref[...].T


def _pack_table(token_t):
    grid = pl.cdiv(VOCAB, VBLK)
    return pl.pallas_call(
        _pack_body,
        grid=(grid,),
        in_specs=[pl.BlockSpec((D, VBLK), lambda j: (0, j))],
        out_specs=pl.BlockSpec((VBLK, 2 * D), lambda j: (j, 0)),
        out_shape=jax.ShapeDtypeStruct((VOCAB, 2 * D), jnp.float32),
    )(token_t)


def _sc_body(tok, idx, pos, out, idx_v, pos_v, b0, b1, b2, b3,
             o0, o1, o2, o3, g0, g1, g2, g3, s0, s1, s2, s3):
    bufs = (b0, b1, b2, b3)
    obufs = (o0, o1, o2, o3)
    gsem = (g0, g1, g2, g3)
    ssem = (s0, s1, s2, s3)
    wid = lax.axis_index("s") * NC + lax.axis_index("c")
    row0 = wid * CPW
    out0 = wid * CPW * CH

    pltpu.sync_copy(idx.at[pl.ds(row0, CPW)], idx_v)
    pltpu.sync_copy(pos, pos_v)

    def gather(s, b):
        pltpu.async_copy(tok.at[idx_v.at[s]], bufs[b], gsem[b])

    def wait_gather(s, b):
        pltpu.make_async_copy(tok.at[idx_v.at[s]], bufs[b], gsem[b]).wait()

    def store(s, b):
        pltpu.async_copy(obufs[b], out.at[pl.ds(out0 + s * CH, CH)], ssem[b])

    def wait_store(s, b):
        pltpu.make_async_copy(
            obufs[b], out.at[pl.ds(out0 + s * CH, CH)], ssem[b]).wait()

    for s in range(NBUF - 1):
        gather(s, s)

    def group(i, carry):
        g = i * NBUF
        for b in range(NBUF):
            s = g + b
            wait_gather(s, b)

            off = lax.rem(s, T // CH) * CH

            def addpos(r, c, _b=b, _off=off):
                for v in range(VPR):
                    sl = pl.ds(v * LANES, LANES)
                    obufs[_b][r, sl] = bufs[_b][r, sl] + pos_v[_off + r, sl]
                return c
            lax.fori_loop(0, CH, addpos, 0, unroll=2)

            t = s + NBUF - 1
            tb = (b + NBUF - 1) % NBUF

            @pl.when(t < CPW)
            def _():
                @pl.when(t >= NBUF)
                def _():
                    wait_store(t - NBUF, tb)
                gather(t, tb)

            store(s, b)
        return carry

    lax.fori_loop(0, CPW // NBUF, group, 0)

    for s in range(CPW - NBUF, CPW):
        wait_store(s, s % NBUF)


def _epi_body(rows_ref, out_ref):
    for tt in range(TBLK):
        out_ref[tt] = rows_ref[:, tt, :].T


def _epilogue(rows):
    rows3 = rows.reshape(B, T, D)
    return pl.pallas_call(
        _epi_body,
        grid=(T // TBLK,),
        in_specs=[pl.BlockSpec((B, TBLK, D), lambda j: (0, j, 0))],
        out_specs=pl.BlockSpec((TBLK, D, B), lambda j: (j, 0, 0)),
        out_shape=jax.ShapeDtypeStruct((T, D, B), jnp.float32),
    )(rows3)


def kernel(input_ids, token_emb, pos_emb):
    ids = input_ids.reshape(NW * CPW, CH).astype(jnp.int32)
    tok = _pack_table(token_emb.T)
    mesh = plsc.VectorSubcoreMesh(core_axis_name="c", subcore_axis_name="s")
    rows = pl.kernel(
        _sc_body,
        out_type=jax.ShapeDtypeStruct((B * T, D), jnp.float32),
        mesh=mesh,
        compiler_params=pltpu.CompilerParams(use_tc_tiling_on_sc=True),
        scratch_types=[
            pltpu.VMEM((CPW, CH), jnp.int32),
            pltpu.VMEM((MAX_LEN, D), jnp.float32),
        ] + [pltpu.VMEM((CH, 2 * D), jnp.float32) for _ in range(NBUF)]
          + [pltpu.VMEM((CH, D), jnp.float32) for _ in range(NBUF)]
          + [pltpu.SemaphoreType.DMA for _ in range(2 * NBUF)],
    )(tok, ids, pos_emb)
    out_tdb = _epilogue(rows)
    return out_tdb.transpose(2, 0, 1)


# speedup vs baseline: 1.2224x; 1.2224x over previous
import jax
import jax.numpy as jnp
from jax import lax
from jax.experimental import pallas as pl
from jax.experimental.pallas import tpu as pltpu
from jax.experimental.pallas import tpu_sc as plsc

VOCAB = 1000000
MAX_LEN = 200
D = 64
B = 1024
T = 200

NC = 2
NS = 16
NW = NC * NS
CH = 40
CPW = (B * T) // (NW * CH)
NBUF = 4
LANES = 16
VPR = D // LANES

VBLK = 4096
NPBLK = 123
OFFSET = VBLK * NPBLK
PROWS = OFFSET
NIBLK = pl.cdiv(VOCAB, VBLK) - 1
TBLK = 8


def _pack_body(ta_ref, tb_ref, out_ref):
    out_ref[:, pl.ds(0, D)] = ta_ref[...].T
    out_ref[:, pl.ds(D, D)] = tb_ref[...].T


def _pack_table(token_t):
    return pl.pallas_call(
        _pack_body,
        grid=(NPBLK,),
        in_specs=[
            pl.BlockSpec((D, VBLK), lambda j: (0, j)),
            pl.BlockSpec((D, VBLK), lambda j: (0, jnp.minimum(j + NPBLK,
                                                              NIBLK))),
        ],
        out_specs=pl.BlockSpec((VBLK, 2 * D), lambda j: (j, 0)),
        out_shape=jax.ShapeDtypeStruct((PROWS, 2 * D), jnp.float32),
    )(token_t, token_t)


def _sc_body(tok, idx, out, idx_v, b0, b1, b2, b3,
             g0, g1, g2, g3, s0, s1, s2, s3):
    bufs = (b0, b1, b2, b3)
    gsem = (g0, g1, g2, g3)
    ssem = (s0, s1, s2, s3)
    wid = lax.axis_index("s") * NC + lax.axis_index("c")
    row0 = wid * CPW
    out0 = wid * CPW * CH

    pltpu.sync_copy(idx.at[pl.ds(row0, CPW)], idx_v)

    def gather(s, b):
        pltpu.async_copy(tok.at[idx_v.at[s]], bufs[b], gsem[b])

    def wait_gather(s, b):
        pltpu.make_async_copy(tok.at[idx_v.at[s]], bufs[b], gsem[b]).wait()

    def store(s, b):
        pltpu.async_copy(bufs[b], out.at[pl.ds(out0 + s * CH, CH)], ssem[b])

    def wait_store(s, b):
        pltpu.make_async_copy(
            bufs[b], out.at[pl.ds(out0 + s * CH, CH)], ssem[b]).wait()

    for s in range(NBUF - 1):
        gather(s, s)

    def group(i, carry):
        g = i * NBUF
        for b in range(NBUF):
            s = g + b
            wait_gather(s, b)

            t = s + NBUF - 1
            tb = (b + NBUF - 1) % NBUF

            @pl.when(t < CPW)
            def _():
                @pl.when(t >= NBUF)
                def _():
                    wait_store(t - NBUF, tb)
                gather(t, tb)

            store(s, b)
        return carry

    lax.fori_loop(0, CPW // NBUF, group, 0)

    for s in range(CPW - NBUF, CPW):
        wait_store(s, s % NBUF)


def _epi_body(rows_ref, par_ref, pos_ref, out_ref):
    for tt in range(TBLK):
        x = rows_ref[:, tt, :]
        lo = x[:, 0:D]
        hi = x[:, D:2 * D]
        pr = par_ref[tt, :]
        xx = jnp.where(pr[:, None] != 0, hi, lo)
        p = pos_ref[pl.ds(tt, 1), :]
        out_ref[tt] = (xx + p).T


def _epilogue(rows, par_t, pos_emb):
    rows3 = rows.reshape(B, T, 2 * D)
    return pl.pallas_call(
        _epi_body,
        grid=(T // TBLK,),
        in_specs=[
            pl.BlockSpec((B, TBLK, 2 * D), lambda j: (0, j, 0)),
            pl.BlockSpec((TBLK, B), lambda j: (j, 0)),
            pl.BlockSpec((TBLK, D), lambda j: (j, 0)),
        ],
        out_specs=pl.BlockSpec((TBLK, D, B), lambda j: (j, 0, 0)),
        out_shape=jax.ShapeDtypeStruct((T, D, B), jnp.float32),
    )(rows3, par_t, pos_emb)


def kernel(input_ids, token_emb, pos_emb):
    ids = input_ids.reshape(NW * CPW, CH).astype(jnp.int32)
    tok = _pack_table(token_emb.T)
    idx = jnp.where(ids >= OFFSET, ids - OFFSET, ids)
    par_t = (input_ids >= OFFSET).astype(jnp.int32).T
    mesh = plsc.VectorSubcoreMesh(core_axis_name="c", subcore_axis_name="s")
    rows = pl.kernel(
        _sc_body,
        out_type=jax.ShapeDtypeStruct((B * T, 2 * D), jnp.float32),
        mesh=mesh,
        compiler_params=pltpu.CompilerParams(use_tc_tiling_on_sc=True),
        scratch_types=[
            pltpu.VMEM((CPW, CH), jnp.int32),
        ] + [pltpu.VMEM((CH, 2 * D), jnp.float32) for _ in range(NBUF)]
          + [pltpu.SemaphoreType.DMA for _ in range(2 * NBUF)],
    )(tok, idx)
    out_tdb = _epilogue(rows, par_t, pos_emb)
    return out_tdb.transpose(2, 0, 1)


# speedup vs baseline: 1.3181x; 1.0783x over previous
import jax
import jax.numpy as jnp
from jax import lax
from jax.experimental import pallas as pl
from jax.experimental.pallas import tpu as pltpu
from jax.experimental.pallas import tpu_sc as plsc

VOCAB = 1000000
MAX_LEN = 200
D = 64
B = 1024
T = 200

NC = 2
NS = 16
NW = NC * NS
CH = 40
CPW = (B * T) // (NW * CH)
NBUF = 4
LANES = 16
VPR = D // LANES

VBLK = 8192
NPBLK = 62
OFFSET = VBLK * NPBLK
PROWS = OFFSET
NIBLK = pl.cdiv(VOCAB, VBLK) - 1
TBLK = 8


def _pack_body(ta_ref, tb_ref, out_ref):
    out_ref[:, pl.ds(0, D)] = ta_ref[...].T
    out_ref[:, pl.ds(D, D)] = tb_ref[...].T


def _pack_table(token_t):
    return pl.pallas_call(
        _pack_body,
        grid=(NPBLK,),
        in_specs=[
            pl.BlockSpec((D, VBLK), lambda j: (0, j)),
            pl.BlockSpec((D, VBLK), lambda j: (0, jnp.minimum(j + NPBLK,
                                                              NIBLK))),
        ],
        out_specs=pl.BlockSpec((VBLK, 2 * D), lambda j: (j, 0)),
        out_shape=jax.ShapeDtypeStruct((PROWS, 2 * D), jnp.float32),
    )(token_t, token_t)


def _sc_body(tok, idx, out, idx_v, b0, b1, b2, b3,
             g0, g1, g2, g3, s0, s1, s2, s3):
    bufs = (b0, b1, b2, b3)
    gsem = (g0, g1, g2, g3)
    ssem = (s0, s1, s2, s3)
    wid = lax.axis_index("s") * NC + lax.axis_index("c")
    row0 = wid * CPW
    out0 = wid * CPW * CH

    pltpu.sync_copy(idx.at[pl.ds(row0, CPW)], idx_v)

    def gather(s, b):
        pltpu.async_copy(tok.at[idx_v.at[s]], bufs[b], gsem[b])

    def wait_gather(s, b):
        pltpu.make_async_copy(tok.at[idx_v.at[s]], bufs[b], gsem[b]).wait()

    def store(s, b):
        pltpu.async_copy(bufs[b], out.at[pl.ds(out0 + s * CH, CH)], ssem[b])

    def wait_store(s, b):
        pltpu.make_async_copy(
            bufs[b], out.at[pl.ds(out0 + s * CH, CH)], ssem[b]).wait()

    for s in range(NBUF - 1):
        gather(s, s)

    def group(i, carry):
        g = i * NBUF
        for b in range(NBUF):
            s = g + b
            wait_gather(s, b)

            t = s + NBUF - 1
            tb = (b + NBUF - 1) % NBUF

            @pl.when(t < CPW)
            def _():
                @pl.when(t >= NBUF)
                def _():
                    wait_store(t - NBUF, tb)
                gather(t, tb)

            store(s, b)
        return carry

    lax.fori_loop(0, CPW // NBUF, group, 0)

    for s in range(CPW - NBUF, CPW):
        wait_store(s, s % NBUF)


def _epi_body(rows_ref, par_ref, pos_ref, out_ref):
    for tt in range(TBLK):
        x = rows_ref[:, tt, :]
        lo = x[:, 0:D]
        hi = x[:, D:2 * D]
        pr = par_ref[tt, :]
        xx = jnp.where(pr[:, None] != 0, hi, lo)
        p = pos_ref[pl.ds(tt, 1), :]
        out_ref[tt] = (xx + p).T


def _epilogue(rows, par_t, pos_emb):
    rows3 = rows.reshape(B, T, 2 * D)
    return pl.pallas_call(
        _epi_body,
        grid=(T // TBLK,),
        in_specs=[
            pl.BlockSpec((B, TBLK, 2 * D), lambda j: (0, j, 0)),
            pl.BlockSpec((TBLK, B), lambda j: (j, 0)),
            pl.BlockSpec((TBLK, D), lambda j: (j, 0)),
        ],
        out_specs=pl.BlockSpec((TBLK, D, B), lambda j: (j, 0, 0)),
        out_shape=jax.ShapeDtypeStruct((T, D, B), jnp.float32),
    )(rows3, par_t, pos_emb)


def kernel(input_ids, token_emb, pos_emb):
    ids = input_ids.reshape(NW * CPW, CH).astype(jnp.int32)
    tok = _pack_table(token_emb.T)
    idx = jnp.where(ids >= OFFSET, ids - OFFSET, ids)
    par_t = (input_ids >= OFFSET).astype(jnp.int32).T
    mesh = plsc.VectorSubcoreMesh(core_axis_name="c", subcore_axis_name="s")
    rows = pl.kernel(
        _sc_body,
        out_type=jax.ShapeDtypeStruct((B * T, 2 * D), jnp.float32),
        mesh=mesh,
        compiler_params=pltpu.CompilerParams(use_tc_tiling_on_sc=True),
        scratch_types=[
            pltpu.VMEM((CPW, CH), jnp.int32),
        ] + [pltpu.VMEM((CH, 2 * D), jnp.float32) for _ in range(NBUF)]
          + [pltpu.SemaphoreType.DMA for _ in range(2 * NBUF)],
    )(tok, idx)
    out_tdb = _epilogue(rows, par_t, pos_emb)
    return out_tdb.transpose(2, 0, 1)


# speedup vs baseline: 1.3612x; 1.0327x over previous
import jax
import jax.numpy as jnp
from jax import lax
from jax.experimental import pallas as pl
from jax.experimental.pallas import tpu as pltpu
from jax.experimental.pallas import tpu_sc as plsc

VOCAB = 1000000
MAX_LEN = 200
D = 64
B = 1024
T = 200

NC = 2
NS = 16
NW = NC * NS
CH = 40
CPW = (B * T) // (NW * CH)
NBUF = 4
LANES = 16
VPR = D // LANES

VBLK = 16384
NPBLK = 31
OFFSET = VBLK * NPBLK
PROWS = OFFSET
NIBLK = pl.cdiv(VOCAB, VBLK) - 1
TBLK = 8


def _pack_body(ta_ref, tb_ref, out_ref):
    out_ref[:, pl.ds(0, D)] = ta_ref[...].T
    out_ref[:, pl.ds(D, D)] = tb_ref[...].T


def _pack_table(token_t):
    return pl.pallas_call(
        _pack_body,
        grid=(NPBLK,),
        in_specs=[
            pl.BlockSpec((D, VBLK), lambda j: (0, j)),
            pl.BlockSpec((D, VBLK), lambda j: (0, jnp.minimum(j + NPBLK,
                                                              NIBLK))),
        ],
        out_specs=pl.BlockSpec((VBLK, 2 * D), lambda j: (j, 0)),
        out_shape=jax.ShapeDtypeStruct((PROWS, 2 * D), jnp.float32),
    )(token_t, token_t)


def _sc_body(tok, idx, out, idx_v, b0, b1, b2, b3,
             g0, g1, g2, g3, s0, s1, s2, s3):
    bufs = (b0, b1, b2, b3)
    gsem = (g0, g1, g2, g3)
    ssem = (s0, s1, s2, s3)
    wid = lax.axis_index("s") * NC + lax.axis_index("c")
    row0 = wid * CPW
    out0 = wid * CPW * CH

    pltpu.sync_copy(idx.at[pl.ds(row0, CPW)], idx_v)

    def gather(s, b):
        pltpu.async_copy(tok.at[idx_v.at[s]], bufs[b], gsem[b])

    def wait_gather(s, b):
        pltpu.make_async_copy(tok.at[idx_v.at[s]], bufs[b], gsem[b]).wait()

    def store(s, b):
        pltpu.async_copy(bufs[b], out.at[pl.ds(out0 + s * CH, CH)], ssem[b])

    def wait_store(s, b):
        pltpu.make_async_copy(
            bufs[b], out.at[pl.ds(out0 + s * CH, CH)], ssem[b]).wait()

    for s in range(NBUF - 1):
        gather(s, s)

    def group(i, carry):
        g = i * NBUF
        for b in range(NBUF):
            s = g + b
            wait_gather(s, b)

            t = s + NBUF - 1
            tb = (b + NBUF - 1) % NBUF

            @pl.when(t < CPW)
            def _():
                @pl.when(t >= NBUF)
                def _():
                    wait_store(t - NBUF, tb)
                gather(t, tb)

            store(s, b)
        return carry

    lax.fori_loop(0, CPW // NBUF, group, 0)

    for s in range(CPW - NBUF, CPW):
        wait_store(s, s % NBUF)


def _epi_body(rows_ref, par_ref, pos_ref, out_ref):
    for tt in range(TBLK):
        x = rows_ref[:, tt, :]
        lo = x[:, 0:D]
        hi = x[:, D:2 * D]
        pr = par_ref[tt, :]
        xx = jnp.where(pr[:, None] != 0, hi, lo)
        p = pos_ref[pl.ds(tt, 1), :]
        out_ref[tt] = (xx + p).T


def _epilogue(rows, par_t, pos_emb):
    rows3 = rows.reshape(B, T, 2 * D)
    return pl.pallas_call(
        _epi_body,
        grid=(T // TBLK,),
        in_specs=[
            pl.BlockSpec((B, TBLK, 2 * D), lambda j: (0, j, 0)),
            pl.BlockSpec((TBLK, B), lambda j: (j, 0)),
            pl.BlockSpec((TBLK, D), lambda j: (j, 0)),
        ],
        out_specs=pl.BlockSpec((TBLK, D, B), lambda j: (j, 0, 0)),
        out_shape=jax.ShapeDtypeStruct((T, D, B), jnp.float32),
    )(rows3, par_t, pos_emb)


def kernel(input_ids, token_emb, pos_emb):
    ids = input_ids.reshape(NW * CPW, CH).astype(jnp.int32)
    tok = _pack_table(token_emb.T)
    idx = jnp.where(ids >= OFFSET, ids - OFFSET, ids)
    par_t = (input_ids >= OFFSET).astype(jnp.int32).T
    mesh = plsc.VectorSubcoreMesh(core_axis_name="c", subcore_axis_name="s")
    rows = pl.kernel(
        _sc_body,
        out_type=jax.ShapeDtypeStruct((B * T, 2 * D), jnp.float32),
        mesh=mesh,
        compiler_params=pltpu.CompilerParams(use_tc_tiling_on_sc=True),
        scratch_types=[
            pltpu.VMEM((CPW, CH), jnp.int32),
        ] + [pltpu.VMEM((CH, 2 * D), jnp.float32) for _ in range(NBUF)]
          + [pltpu.SemaphoreType.DMA for _ in range(2 * NBUF)],
    )(tok, idx)
    out_tdb = _epilogue(rows, par_t, pos_emb)
    return out_tdb.transpose(2, 0, 1)


# speedup vs baseline: 1.4871x; 1.0925x over previous
import jax
import jax.numpy as jnp
from jax import lax
from jax.experimental import pallas as pl
from jax.experimental.pallas import tpu as pltpu
from jax.experimental.pallas import tpu_sc as plsc

VOCAB = 1000000
MAX_LEN = 200
D = 64
B = 1024
T = 200

NC = 2
NS = 16
NW = NC * NS
CH = 40
CPW = (B * T) // (NW * CH)
NBUF = 4
LANES = 16
VPR = D // LANES

VBLK = 16384
NPBLK = 31
OFFSET = VBLK * NPBLK
PROWS = OFFSET
NIBLK = pl.cdiv(VOCAB, VBLK) - 1
TBLK = 8


def _pack_body(ta_ref, tb_ref, out_ref):
    out_ref[:, pl.ds(0, D)] = ta_ref[...].T
    out_ref[:, pl.ds(D, D)] = tb_ref[...].T


def _pack_table(token_t):
    return pl.pallas_call(
        _pack_body,
        grid=(NPBLK,),
        in_specs=[
            pl.BlockSpec((D, VBLK), lambda j: (0, j)),
            pl.BlockSpec((D, VBLK), lambda j: (0, jnp.minimum(j + NPBLK,
                                                              NIBLK))),
        ],
        out_specs=pl.BlockSpec((VBLK, 2 * D), lambda j: (j, 0)),
        out_shape=jax.ShapeDtypeStruct((PROWS, 2 * D), jnp.float32),
    )(token_t, token_t)


def _sc_body(tok, idx, out, idx_v, b0, b1, b2, b3,
             g0, g1, g2, g3, s0, s1, s2, s3):
    bufs = (b0, b1, b2, b3)
    gsem = (g0, g1, g2, g3)
    ssem = (s0, s1, s2, s3)
    wid = lax.axis_index("s") * NC + lax.axis_index("c")
    row0 = wid * CPW
    out0 = wid * CPW * CH

    pltpu.sync_copy(idx.at[pl.ds(row0, CPW)], idx_v)

    def gather(s, b):
        pltpu.async_copy(tok.at[idx_v.at[s]], bufs[b], gsem[b])

    def wait_gather(s, b):
        pltpu.make_async_copy(tok.at[idx_v.at[s]], bufs[b], gsem[b]).wait()

    def store(s, b):
        pltpu.async_copy(bufs[b], out.at[pl.ds(out0 + s * CH, CH)], ssem[b])

    def wait_store(s, b):
        pltpu.make_async_copy(
            bufs[b], out.at[pl.ds(out0 + s * CH, CH)], ssem[b]).wait()

    for s in range(NBUF - 1):
        gather(s, s)

    def group(i, carry):
        g = i * NBUF
        for b in range(NBUF):
            s = g + b
            wait_gather(s, b)

            t = s + NBUF - 1
            tb = (b + NBUF - 1) % NBUF

            @pl.when(t < CPW)
            def _():
                @pl.when(t >= NBUF)
                def _():
                    wait_store(t - NBUF, tb)
                gather(t, tb)

            store(s, b)
        return carry

    lax.fori_loop(0, CPW // NBUF, group, 0)

    for s in range(CPW - NBUF, CPW):
        wait_store(s, s % NBUF)


def _epi_body(rows_ref, par_ref, pos_ref, out_ref):
    for tt in range(TBLK):
        x = rows_ref[:, tt, :]
        xt = x.T
        lo = xt[0:D, :]
        hi = xt[D:2 * D, :]
        pr = par_ref[pl.ds(tt, 1), :]
        p = pos_ref[pl.ds(tt, 1), :]
        out_ref[tt] = jnp.where(pr != 0, hi, lo) + p.T


def _epilogue(rows, par_t, pos_emb):
    rows3 = rows.reshape(B, T, 2 * D)
    return pl.pallas_call(
        _epi_body,
        grid=(T // TBLK,),
        in_specs=[
            pl.BlockSpec((B, TBLK, 2 * D), lambda j: (0, j, 0)),
            pl.BlockSpec((TBLK, B), lambda j: (j, 0)),
            pl.BlockSpec((TBLK, D), lambda j: (j, 0)),
        ],
        out_specs=pl.BlockSpec((TBLK, D, B), lambda j: (j, 0, 0)),
        out_shape=jax.ShapeDtypeStruct((T, D, B), jnp.float32),
    )(rows3, par_t, pos_emb)


def kernel(input_ids, token_emb, pos_emb):
    ids = input_ids.reshape(NW * CPW, CH).astype(jnp.int32)
    tok = _pack_table(token_emb.T)
    idx = jnp.where(ids >= OFFSET, ids - OFFSET, ids)
    par_t = (input_ids >= OFFSET).astype(jnp.int32).T
    mesh = plsc.VectorSubcoreMesh(core_axis_name="c", subcore_axis_name="s")
    rows = pl.kernel(
        _sc_body,
        out_type=jax.ShapeDtypeStruct((B * T, 2 * D), jnp.float32),
        mesh=mesh,
        compiler_params=pltpu.CompilerParams(use_tc_tiling_on_sc=True),
        scratch_types=[
            pltpu.VMEM((CPW, CH), jnp.int32),
        ] + [pltpu.VMEM((CH, 2 * D), jnp.float32) for _ in range(NBUF)]
          + [pltpu.SemaphoreType.DMA for _ in range(2 * NBUF)],
    )(tok, idx)
    out_tdb = _epilogue(rows, par_t, pos_emb)
    return out_tdb.transpose(2, 0, 1)
